# baseline (speedup 1.0000x reference)
import functools

import jax
import jax.numpy as jnp
from jax import lax
from jax.experimental import pallas as pl
from jax.experimental.pallas import tpu as pltpu
from jax.experimental.pallas import tpu_sc as plsc

_E = 64
_Q = 1.2
_EPS = 1e-5
_ENT_TH = 2.5
_TOP_P = 0.75

_N = 65536
_NC, _NS, _L = 2, 16, 16
_NW = _NC * _NS
_W_SC = 8192
_TPW = _W_SC // _NW
_CH = 256
_NCH = _TPW // _CH
_NG = _CH // _L
_PR = 2 * _E + 4
_TB = 2048


def _sc_body(x_hbm, w_hbm, out_hbm, xb, wb, pbuf, acc):
    wid = lax.axis_index("s") * _NC + lax.axis_index("c")
    zeros = jnp.zeros((_L,), jnp.float32)

    def _zinit(j, carry):
        acc[j, :] = zeros
        return carry

    lax.fori_loop(0, _PR, _zinit, 0)

    def _chunk_body(std0):

        def _group(g, std):
            S, T, D = std
            sl = pl.ds(g * _L, _L)
            ninf = jnp.full((_L,), -1e30, jnp.float32)

            def _p1(blk, carry):
                m, m2x = carry
                for k in range(8):
                    x = xb[blk * 8 + k, sl]
                    m2x = jnp.maximum(m2x, jnp.minimum(m, x))
                    m = jnp.maximum(m, x)
                return m, m2x

            m, m2x = lax.fori_loop(0, 8, _p1, (ninf, ninf))

            def _p2(blk, z):
                for k in range(8):
                    j = blk * 8 + k
                    e = jnp.exp(xb[j, sl] - m)
                    pbuf[j, sl] = e
                    z = z + e
                return z

            z = lax.fori_loop(0, 8, _p2, zeros)
            rz = 1.0 / z

            y0 = jnp.where(z >= 20.0855, 3.5,
                           jnp.where(z >= 7.3891, 2.5,
                                     jnp.where(z >= 2.7183, 1.5, 0.5)))

            def _newton(_, y):
                return y - 1.0 + z * jnp.exp(-y)

            lnz = lax.fori_loop(0, 5, _newton, y0)
            k12 = _Q * (m + lnz)
            e2 = jnp.exp(m2x - m)

            def _p4(blk, carry):
                S, pqs = carry
                for k in range(8):
                    j = blk * 8 + k
                    ev = pbuf[j, sl]
                    S = S + jnp.maximum(ev * rz, _EPS)
                    pqs = pqs + jnp.maximum(jnp.exp(_Q * xb[j, sl] - k12),
                                            1e-6)
                return S, pqs

            S, pqs = lax.fori_loop(0, 8, _p4, (S, zeros))
            T = T + pqs
            high = ((1.0 - pqs) / (_Q - 1.0)) >= _ENT_TH

            thr = _TOP_P * z

            def _bis(_, lh):
                lo, hi = lh
                u = 0.5 * (lo + hi)

                def _gsum(blk, gs):
                    for k in range(8):
                        ev = pbuf[blk * 8 + k, sl]
                        gs = gs + jnp.where(ev >= u, ev, 0.0)
                    return gs

                gs = lax.fori_loop(0, 8, _gsum, zeros)
                ok = gs <= thr
                return jnp.where(ok, lo, u), jnp.where(ok, u, hi)

            _, hi = lax.fori_loop(
                0, 16, _bis, (zeros, jnp.full((_L,), 2.0, jnp.float32)))

            wv = wb[sl]
            rzw = rz * wv

            def _p6(blk, carry):
                for k in range(8):
                    j = blk * 8 + k
                    ev = pbuf[j, sl]
                    keep = high | (ev >= e2) | (ev >= hi)
                    rwv = jnp.where(keep, ev, 0.0)
                    plsc.addupdate(acc.at[j, :], rwv * rzw)
                    plsc.addupdate(acc.at[_E + j, :], ev * rzw)
                return carry

            lax.fori_loop(0, 8, _p6, 0)
            return (S, T, D + wv)

        return lax.fori_loop(0, _NG, _group, std0)

    std = (zeros, zeros, zeros)
    for c in range(_NCH):
        pltpu.sync_copy(x_hbm.at[wid, c], xb)
        pltpu.sync_copy(w_hbm.at[wid, c], wb)
        std = _chunk_body(std)
    S, T, D = std

    acc[2 * _E, :] = S
    acc[2 * _E + 1, :] = T
    acc[2 * _E + 2, :] = D
    acc[2 * _E + 3, :] = zeros
    pltpu.sync_copy(acc, out_hbm.at[wid])


def _tc_body(nb, x_ref, w_ref, out_ref, accA, accB, accSTD):
    b = pl.program_id(0)

    @pl.when(b == 0)
    def _init():
        accA[...] = jnp.zeros_like(accA)
        accB[...] = jnp.zeros_like(accB)
        accSTD[...] = jnp.zeros_like(accSTD)

    x = x_ref[...].T
    w = w_ref[...]
    mx = jnp.max(x, axis=0, keepdims=True)
    e = jnp.exp(x - mx)
    z = jnp.sum(e, axis=0, keepdims=True)
    p = e / z

    pc = jnp.maximum(p, _EPS)
    pq = jnp.maximum(jnp.exp(_Q * (x - mx - jnp.log(z))), 1e-6)
    sum_pq_tok = jnp.sum(pq, axis=0, keepdims=True)
    ent = (1.0 - sum_pq_tok) / (_Q - 1.0)
    high = ent >= _ENT_TH

    m1 = jnp.max(p, axis=0, keepdims=True)
    m2 = jnp.max(jnp.where(p < m1, p, -1.0), axis=0, keepdims=True)

    lo = jnp.zeros_like(m1)
    hi = jnp.full_like(m1, 2.0)
    for _ in range(10):
        u = 0.5 * (lo + hi)
        gs = jnp.sum(jnp.where(p >= u, p, 0.0), axis=0, keepdims=True)
        ok = gs <= _TOP_P
        lo = jnp.where(ok, lo, u)
        hi = jnp.where(ok, u, hi)

    keep = high | (p >= m2) | (p >= hi)
    rw = jnp.where(keep, p, 0.0)

    accA[...] += jnp.sum(rw * w, axis=1, keepdims=True)
    accB[...] += jnp.sum(p * w, axis=1, keepdims=True)
    accSTD[0:1, :] += jnp.sum(pc)
    accSTD[1:2, :] += jnp.sum(pq)
    accSTD[2:3, :] += jnp.sum(w)

    @pl.when(b == nb - 1)
    def _fin():
        out_ref[0:_E, :] = accA[...]
        out_ref[_E:2 * _E, :] = accB[...]
        out_ref[2 * _E:2 * _E + 3, :] = accSTD[...]
        out_ref[2 * _E + 3:, :] = jnp.zeros_like(out_ref[2 * _E + 3:, :])


def _combine_body(p_sc_ref, p_tc_ref, out_ref):
    pm = jnp.sum(p_sc_ref[...], axis=0)
    q = p_tc_ref[...]
    a = jnp.sum(pm[0:_E, :], axis=1, keepdims=True) + q[0:_E, :]
    b = (jnp.sum(pm[_E:2 * _E, :], axis=1, keepdims=True)
         + q[_E:2 * _E, :])
    s = (jnp.sum(pm[2 * _E:2 * _E + 1, :], axis=1, keepdims=True)
         + q[2 * _E:2 * _E + 1, :])
    t = (jnp.sum(pm[2 * _E + 1:2 * _E + 2, :], axis=1, keepdims=True)
         + q[2 * _E + 1:2 * _E + 2, :])
    d = (jnp.sum(pm[2 * _E + 2:2 * _E + 3, :], axis=1, keepdims=True)
         + q[2 * _E + 2:2 * _E + 3, :])
    ent = (1.0 - t / (s ** _Q)) / (_Q - 1.0)
    lb = _E * jnp.sum(a * b, axis=0, keepdims=True) / (d * d)
    out_ref[...] = 0.001 * ent + 0.001 * lb


def kernel(gate_logits, attention_mask):
    n, e = gate_logits.shape
    bsz, seq = attention_mask.shape
    layers = n // (bsz * seq)

    wrow = jnp.broadcast_to(
        attention_mask.reshape(-1)[None, :], (layers, bsz * seq)
    ).reshape(1, n).astype(jnp.float32)

    x_r = gate_logits[:_W_SC].reshape(_NW, _NCH, _CH, e).transpose(0, 1, 3, 2)
    w_r = wrow[0, :_W_SC].reshape(_NW, _NCH, _CH)

    mesh = plsc.VectorSubcoreMesh(
        core_axis_name="c", subcore_axis_name="s",
        num_cores=_NC, num_subcores=_NS)
    partials_sc = pl.kernel(
        _sc_body,
        out_type=jax.ShapeDtypeStruct((_NW, _PR, _L), jnp.float32),
        mesh=mesh,
        scratch_types=[
            pltpu.VMEM((_E, _CH), jnp.float32),
            pltpu.VMEM((_CH,), jnp.float32),
            pltpu.VMEM((_E, _CH), jnp.float32),
            pltpu.VMEM((_PR, _L), jnp.float32),
        ],
    )(x_r, w_r)

    nb = (n - _W_SC) // _TB
    off = _W_SC // _TB
    part_tc = pl.pallas_call(
        functools.partial(_tc_body, nb),
        grid=(nb,),
        in_specs=[
            pl.BlockSpec((_TB, e), lambda i: (i + off, 0)),
            pl.BlockSpec((1, _TB), lambda i: (0, i + off)),
        ],
        out_specs=pl.BlockSpec((_PR, 1), lambda i: (0, 0)),
        out_shape=jax.ShapeDtypeStruct((_PR, 1), jnp.float32),
        scratch_shapes=[
            pltpu.VMEM((_E, 1), jnp.float32),
            pltpu.VMEM((_E, 1), jnp.float32),
            pltpu.VMEM((3, 1), jnp.float32),
        ],
        compiler_params=pltpu.CompilerParams(
            dimension_semantics=("arbitrary",),
        ),
    )(gate_logits, wrow)

    loss = pl.pallas_call(
        _combine_body,
        out_shape=jax.ShapeDtypeStruct((1, 1), jnp.float32),
    )(partials_sc, part_tc)
    return loss.reshape(())


# speedup vs baseline: 1.2254x; 1.2254x over previous
import functools

import jax
import jax.numpy as jnp
from jax import lax
from jax.experimental import pallas as pl
from jax.experimental.pallas import tpu as pltpu
from jax.experimental.pallas import tpu_sc as plsc

_E = 64
_Q = 1.2
_EPS = 1e-5
_ENT_TH = 2.5
_TOP_P = 0.75

_N = 65536
_NC, _NS, _L = 2, 16, 16
_NW = _NC * _NS
_W_SC = 8192
_TPW = _W_SC // _NW
_CH = 256
_NCH = _TPW // _CH
_NG = _CH // _L
_PR = 2 * _E + 4
_TB = 1024


def _sc_body(x_hbm, w_hbm, out_hbm, xb, wb, pbuf, acc):
    wid = lax.axis_index("s") * _NC + lax.axis_index("c")
    zeros = jnp.zeros((_L,), jnp.float32)

    def _zinit(j, carry):
        acc[j, :] = zeros
        return carry

    lax.fori_loop(0, _PR, _zinit, 0)

    def _chunk_body(std0):

        def _group(g, std):
            S, T, D = std
            sl = pl.ds(g * _L, _L)
            ninf = jnp.full((_L,), -1e30, jnp.float32)

            def _p1(blk, carry):
                m, m2x = carry
                for k in range(8):
                    x = xb[blk * 8 + k, sl]
                    m2x = jnp.maximum(m2x, jnp.minimum(m, x))
                    m = jnp.maximum(m, x)
                return m, m2x

            m, m2x = lax.fori_loop(0, 8, _p1, (ninf, ninf))

            def _p2(blk, z):
                for k in range(8):
                    j = blk * 8 + k
                    e = jnp.exp(xb[j, sl] - m)
                    pbuf[j, sl] = e
                    z = z + e
                return z

            z = lax.fori_loop(0, 8, _p2, zeros)
            rz = 1.0 / z

            y0 = jnp.where(z >= 20.0855, 3.5,
                           jnp.where(z >= 7.3891, 2.5,
                                     jnp.where(z >= 2.7183, 1.5, 0.5)))

            def _newton(_, y):
                return y - 1.0 + z * jnp.exp(-y)

            lnz = lax.fori_loop(0, 5, _newton, y0)
            k12 = _Q * (m + lnz)
            e2 = jnp.exp(m2x - m)

            def _p4(blk, carry):
                S, pqs = carry
                for k in range(8):
                    j = blk * 8 + k
                    ev = pbuf[j, sl]
                    S = S + jnp.maximum(ev * rz, _EPS)
                    pqs = pqs + jnp.maximum(jnp.exp(_Q * xb[j, sl] - k12),
                                            1e-6)
                return S, pqs

            S, pqs = lax.fori_loop(0, 8, _p4, (S, zeros))
            T = T + pqs
            high = ((1.0 - pqs) / (_Q - 1.0)) >= _ENT_TH

            thr = _TOP_P * z

            def _bis(_, lh):
                lo, hi = lh
                u = 0.5 * (lo + hi)

                def _gsum(blk, gs):
                    for k in range(8):
                        ev = pbuf[blk * 8 + k, sl]
                        gs = gs + jnp.where(ev >= u, ev, 0.0)
                    return gs

                gs = lax.fori_loop(0, 8, _gsum, zeros)
                ok = gs <= thr
                return jnp.where(ok, lo, u), jnp.where(ok, u, hi)

            _, hi = lax.fori_loop(
                0, 16, _bis, (zeros, jnp.full((_L,), 2.0, jnp.float32)))

            wv = wb[sl]
            rzw = rz * wv

            def _p6(blk, carry):
                for k in range(8):
                    j = blk * 8 + k
                    ev = pbuf[j, sl]
                    keep = high | (ev >= e2) | (ev >= hi)
                    rwv = jnp.where(keep, ev, 0.0)
                    plsc.addupdate(acc.at[j, :], rwv * rzw)
                    plsc.addupdate(acc.at[_E + j, :], ev * rzw)
                return carry

            lax.fori_loop(0, 8, _p6, 0)
            return (S, T, D + wv)

        return lax.fori_loop(0, _NG, _group, std0)

    std = (zeros, zeros, zeros)
    for c in range(_NCH):
        pltpu.sync_copy(x_hbm.at[wid, c], xb)
        pltpu.sync_copy(w_hbm.at[wid, c], wb)
        std = _chunk_body(std)
    S, T, D = std

    acc[2 * _E, :] = S
    acc[2 * _E + 1, :] = T
    acc[2 * _E + 2, :] = D
    acc[2 * _E + 3, :] = zeros
    pltpu.sync_copy(acc, out_hbm.at[wid])


def _tc_body(nb, x_ref, w_ref, out_ref, accA, accB, accSTD):
    b = pl.program_id(0)

    @pl.when(b == 0)
    def _init():
        accA[...] = jnp.zeros_like(accA)
        accB[...] = jnp.zeros_like(accB)
        accSTD[...] = jnp.zeros_like(accSTD)

    x = x_ref[...]
    w = w_ref[...]
    mx = jnp.max(x, axis=0, keepdims=True)
    e = jnp.exp(x - mx)
    z = jnp.sum(e, axis=0, keepdims=True)
    p = e / z

    pc = jnp.maximum(p, _EPS)
    pq = jnp.maximum(jnp.exp(_Q * (x - mx - jnp.log(z))), 1e-6)
    sum_pq_tok = jnp.sum(pq, axis=0, keepdims=True)
    ent = (1.0 - sum_pq_tok) / (_Q - 1.0)
    high = ent >= _ENT_TH

    m1 = jnp.max(p, axis=0, keepdims=True)
    m2 = jnp.max(jnp.where(p < m1, p, -1.0), axis=0, keepdims=True)

    lo = jnp.zeros_like(m1)
    hi = jnp.full_like(m1, 2.0)
    for _ in range(10):
        u = 0.5 * (lo + hi)
        gs = jnp.sum(jnp.where(p >= u, p, 0.0), axis=0, keepdims=True)
        ok = gs <= _TOP_P
        lo = jnp.where(ok, lo, u)
        hi = jnp.where(ok, u, hi)

    keep = high | (p >= m2) | (p >= hi)
    rw = jnp.where(keep, p, 0.0)

    accA[...] += jnp.sum(rw * w, axis=1, keepdims=True)
    accB[...] += jnp.sum(p * w, axis=1, keepdims=True)
    accSTD[0:1, :] += jnp.sum(pc)
    accSTD[1:2, :] += jnp.sum(pq)
    accSTD[2:3, :] += jnp.sum(w)

    @pl.when(b == nb - 1)
    def _fin():
        out_ref[0:_E, :] = accA[...]
        out_ref[_E:2 * _E, :] = accB[...]
        out_ref[2 * _E:2 * _E + 3, :] = accSTD[...]
        out_ref[2 * _E + 3:, :] = jnp.zeros_like(out_ref[2 * _E + 3:, :])


def _combine_body(p_sc_ref, p_tc_ref, out_ref):
    pm = jnp.sum(p_sc_ref[...], axis=0)
    q = p_tc_ref[...]
    a = jnp.sum(pm[0:_E, :], axis=1, keepdims=True) + q[0:_E, :]
    b = (jnp.sum(pm[_E:2 * _E, :], axis=1, keepdims=True)
         + q[_E:2 * _E, :])
    s = (jnp.sum(pm[2 * _E:2 * _E + 1, :], axis=1, keepdims=True)
         + q[2 * _E:2 * _E + 1, :])
    t = (jnp.sum(pm[2 * _E + 1:2 * _E + 2, :], axis=1, keepdims=True)
         + q[2 * _E + 1:2 * _E + 2, :])
    d = (jnp.sum(pm[2 * _E + 2:2 * _E + 3, :], axis=1, keepdims=True)
         + q[2 * _E + 2:2 * _E + 3, :])
    ent = (1.0 - t / (s ** _Q)) / (_Q - 1.0)
    lb = _E * jnp.sum(a * b, axis=0, keepdims=True) / (d * d)
    out_ref[...] = 0.001 * ent + 0.001 * lb


def kernel(gate_logits, attention_mask):
    n, e = gate_logits.shape
    bsz, seq = attention_mask.shape
    layers = n // (bsz * seq)

    wrow = jnp.broadcast_to(
        attention_mask.reshape(-1)[None, :], (layers, bsz * seq)
    ).reshape(1, n).astype(jnp.float32)

    x_r = gate_logits[:_W_SC].reshape(_NW, _NCH, _CH, e).transpose(0, 1, 3, 2)
    w_r = wrow[0, :_W_SC].reshape(_NW, _NCH, _CH)

    mesh = plsc.VectorSubcoreMesh(
        core_axis_name="c", subcore_axis_name="s",
        num_cores=_NC, num_subcores=_NS)
    partials_sc = pl.kernel(
        _sc_body,
        out_type=jax.ShapeDtypeStruct((_NW, _PR, _L), jnp.float32),
        mesh=mesh,
        scratch_types=[
            pltpu.VMEM((_E, _CH), jnp.float32),
            pltpu.VMEM((_CH,), jnp.float32),
            pltpu.VMEM((_E, _CH), jnp.float32),
            pltpu.VMEM((_PR, _L), jnp.float32),
        ],
    )(x_r, w_r)

    xt = gate_logits.T
    nb = (n - _W_SC) // _TB
    off = _W_SC // _TB
    part_tc = pl.pallas_call(
        functools.partial(_tc_body, nb),
        grid=(nb,),
        in_specs=[
            pl.BlockSpec((e, _TB), lambda i: (0, i + off)),
            pl.BlockSpec((1, _TB), lambda i: (0, i + off)),
        ],
        out_specs=pl.BlockSpec((_PR, 1), lambda i: (0, 0)),
        out_shape=jax.ShapeDtypeStruct((_PR, 1), jnp.float32),
        scratch_shapes=[
            pltpu.VMEM((_E, 1), jnp.float32),
            pltpu.VMEM((_E, 1), jnp.float32),
            pltpu.VMEM((3, 1), jnp.float32),
        ],
        compiler_params=pltpu.CompilerParams(
            dimension_semantics=("arbitrary",),
        ),
    )(xt, wrow)

    loss = pl.pallas_call(
        _combine_body,
        out_shape=jax.ShapeDtypeStruct((1, 1), jnp.float32),
    )(partials_sc, part_tc)
    return loss.reshape(())


# speedup vs baseline: 1.2864x; 1.0498x over previous
import functools

import jax
import jax.numpy as jnp
from jax import lax
from jax.experimental import pallas as pl
from jax.experimental.pallas import tpu as pltpu
from jax.experimental.pallas import tpu_sc as plsc

_E = 64
_Q = 1.2
_EPS = 1e-5
_ENT_TH = 2.5
_TOP_P = 0.75

_N = 65536
_NC, _NS, _L = 2, 16, 16
_NW = _NC * _NS
_W_SC = 8192
_TPW = _W_SC // _NW
_CH = 256
_NCH = _TPW // _CH
_NG = _CH // _L
_PR = 2 * _E + 4
_TB = 4096


def _sc_body(x_hbm, w_hbm, out_hbm, xb, wb, pbuf, acc):
    wid = lax.axis_index("s") * _NC + lax.axis_index("c")
    zeros = jnp.zeros((_L,), jnp.float32)

    def _zinit(j, carry):
        acc[j, :] = zeros
        return carry

    lax.fori_loop(0, _PR, _zinit, 0)

    def _chunk_body(std0):

        def _group(g, std):
            S, T, D = std
            sl = pl.ds(g * _L, _L)
            ninf = jnp.full((_L,), -1e30, jnp.float32)

            def _p1(blk, carry):
                m, m2x = carry
                for k in range(8):
                    x = xb[blk * 8 + k, sl]
                    m2x = jnp.maximum(m2x, jnp.minimum(m, x))
                    m = jnp.maximum(m, x)
                return m, m2x

            m, m2x = lax.fori_loop(0, 8, _p1, (ninf, ninf))

            def _p2(blk, z):
                for k in range(8):
                    j = blk * 8 + k
                    e = jnp.exp(xb[j, sl] - m)
                    pbuf[j, sl] = e
                    z = z + e
                return z

            z = lax.fori_loop(0, 8, _p2, zeros)
            rz = 1.0 / z

            y0 = jnp.where(z >= 20.0855, 3.5,
                           jnp.where(z >= 7.3891, 2.5,
                                     jnp.where(z >= 2.7183, 1.5, 0.5)))

            def _newton(_, y):
                return y - 1.0 + z * jnp.exp(-y)

            lnz = lax.fori_loop(0, 5, _newton, y0)
            k12 = _Q * (m + lnz)
            e2 = jnp.exp(m2x - m)

            def _p4(blk, carry):
                S, pqs = carry
                for k in range(8):
                    j = blk * 8 + k
                    ev = pbuf[j, sl]
                    S = S + jnp.maximum(ev * rz, _EPS)
                    pqs = pqs + jnp.maximum(jnp.exp(_Q * xb[j, sl] - k12),
                                            1e-6)
                return S, pqs

            S, pqs = lax.fori_loop(0, 8, _p4, (S, zeros))
            T = T + pqs
            high = ((1.0 - pqs) / (_Q - 1.0)) >= _ENT_TH

            thr = _TOP_P * z

            def _bis(_, lh):
                lo, hi = lh
                u = 0.5 * (lo + hi)

                def _gsum(blk, gs):
                    for k in range(8):
                        ev = pbuf[blk * 8 + k, sl]
                        gs = gs + jnp.where(ev >= u, ev, 0.0)
                    return gs

                gs = lax.fori_loop(0, 8, _gsum, zeros)
                ok = gs <= thr
                return jnp.where(ok, lo, u), jnp.where(ok, u, hi)

            _, hi = lax.fori_loop(
                0, 16, _bis, (zeros, jnp.full((_L,), 2.0, jnp.float32)))

            wv = wb[sl]
            rzw = rz * wv

            def _p6(blk, carry):
                for k in range(8):
                    j = blk * 8 + k
                    ev = pbuf[j, sl]
                    keep = high | (ev >= e2) | (ev >= hi)
                    rwv = jnp.where(keep, ev, 0.0)
                    plsc.addupdate(acc.at[j, :], rwv * rzw)
                    plsc.addupdate(acc.at[_E + j, :], ev * rzw)
                return carry

            lax.fori_loop(0, 8, _p6, 0)
            return (S, T, D + wv)

        return lax.fori_loop(0, _NG, _group, std0)

    std = (zeros, zeros, zeros)
    for c in range(_NCH):
        pltpu.sync_copy(x_hbm.at[wid, c], xb)
        pltpu.sync_copy(w_hbm.at[wid, c], wb)
        std = _chunk_body(std)
    S, T, D = std

    acc[2 * _E, :] = S
    acc[2 * _E + 1, :] = T
    acc[2 * _E + 2, :] = D
    acc[2 * _E + 3, :] = zeros
    pltpu.sync_copy(acc, out_hbm.at[wid])


def _tc_body(nb, x_ref, w_ref, out_ref, accA, accB, accSTD):
    b = pl.program_id(0)

    @pl.when(b == 0)
    def _init():
        accA[...] = jnp.zeros_like(accA)
        accB[...] = jnp.zeros_like(accB)
        accSTD[...] = jnp.zeros_like(accSTD)

    x = x_ref[...]
    w = w_ref[...]
    mx = jnp.max(x, axis=0, keepdims=True)
    e = jnp.exp(x - mx)
    z = jnp.sum(e, axis=0, keepdims=True)
    p = e / z

    pc = jnp.maximum(p, _EPS)
    pq = jnp.maximum(jnp.exp(_Q * (x - mx - jnp.log(z))), 1e-6)
    sum_pq_tok = jnp.sum(pq, axis=0, keepdims=True)
    ent = (1.0 - sum_pq_tok) / (_Q - 1.0)
    high = ent >= _ENT_TH

    m1 = jnp.max(p, axis=0, keepdims=True)
    m2 = jnp.max(jnp.where(p < m1, p, -1.0), axis=0, keepdims=True)

    lo = jnp.zeros_like(m1)
    hi = jnp.full_like(m1, 2.0)
    for _ in range(10):
        u = 0.5 * (lo + hi)
        gs = jnp.sum(jnp.where(p >= u, p, 0.0), axis=0, keepdims=True)
        ok = gs <= _TOP_P
        lo = jnp.where(ok, lo, u)
        hi = jnp.where(ok, u, hi)

    keep = high | (p >= m2) | (p >= hi)
    rw = jnp.where(keep, p, 0.0)

    accA[...] += jnp.sum(rw * w, axis=1, keepdims=True)
    accB[...] += jnp.sum(p * w, axis=1, keepdims=True)
    accSTD[0:1, :] += jnp.sum(pc)
    accSTD[1:2, :] += jnp.sum(pq)
    accSTD[2:3, :] += jnp.sum(w)

    @pl.when(b == nb - 1)
    def _fin():
        out_ref[0:_E, :] = accA[...]
        out_ref[_E:2 * _E, :] = accB[...]
        out_ref[2 * _E:2 * _E + 3, :] = accSTD[...]
        out_ref[2 * _E + 3:, :] = jnp.zeros_like(out_ref[2 * _E + 3:, :])


def _combine_body(p_sc_ref, p_tc_ref, out_ref):
    pm = jnp.sum(p_sc_ref[...], axis=0)
    q = p_tc_ref[...]
    a = jnp.sum(pm[0:_E, :], axis=1, keepdims=True) + q[0:_E, :]
    b = (jnp.sum(pm[_E:2 * _E, :], axis=1, keepdims=True)
         + q[_E:2 * _E, :])
    s = (jnp.sum(pm[2 * _E:2 * _E + 1, :], axis=1, keepdims=True)
         + q[2 * _E:2 * _E + 1, :])
    t = (jnp.sum(pm[2 * _E + 1:2 * _E + 2, :], axis=1, keepdims=True)
         + q[2 * _E + 1:2 * _E + 2, :])
    d = (jnp.sum(pm[2 * _E + 2:2 * _E + 3, :], axis=1, keepdims=True)
         + q[2 * _E + 2:2 * _E + 3, :])
    ent = (1.0 - t / (s ** _Q)) / (_Q - 1.0)
    lb = _E * jnp.sum(a * b, axis=0, keepdims=True) / (d * d)
    out_ref[...] = 0.001 * ent + 0.001 * lb


def kernel(gate_logits, attention_mask):
    n, e = gate_logits.shape
    bsz, seq = attention_mask.shape
    layers = n // (bsz * seq)

    wrow = jnp.broadcast_to(
        attention_mask.reshape(-1)[None, :], (layers, bsz * seq)
    ).reshape(1, n).astype(jnp.float32)

    x_r = gate_logits[:_W_SC].reshape(_NW, _NCH, _CH, e).transpose(0, 1, 3, 2)
    w_r = wrow[0, :_W_SC].reshape(_NW, _NCH, _CH)

    mesh = plsc.VectorSubcoreMesh(
        core_axis_name="c", subcore_axis_name="s",
        num_cores=_NC, num_subcores=_NS)
    partials_sc = pl.kernel(
        _sc_body,
        out_type=jax.ShapeDtypeStruct((_NW, _PR, _L), jnp.float32),
        mesh=mesh,
        scratch_types=[
            pltpu.VMEM((_E, _CH), jnp.float32),
            pltpu.VMEM((_CH,), jnp.float32),
            pltpu.VMEM((_E, _CH), jnp.float32),
            pltpu.VMEM((_PR, _L), jnp.float32),
        ],
    )(x_r, w_r)

    xt = gate_logits.T
    nb = (n - _W_SC) // _TB
    off = _W_SC // _TB
    part_tc = pl.pallas_call(
        functools.partial(_tc_body, nb),
        grid=(nb,),
        in_specs=[
            pl.BlockSpec((e, _TB), lambda i: (0, i + off)),
            pl.BlockSpec((1, _TB), lambda i: (0, i + off)),
        ],
        out_specs=pl.BlockSpec((_PR, 1), lambda i: (0, 0)),
        out_shape=jax.ShapeDtypeStruct((_PR, 1), jnp.float32),
        scratch_shapes=[
            pltpu.VMEM((_E, 1), jnp.float32),
            pltpu.VMEM((_E, 1), jnp.float32),
            pltpu.VMEM((3, 1), jnp.float32),
        ],
        compiler_params=pltpu.CompilerParams(
            dimension_semantics=("arbitrary",),
        ),
    )(xt, wrow)

    loss = pl.pallas_call(
        _combine_body,
        out_shape=jax.ShapeDtypeStruct((1, 1), jnp.float32),
    )(partials_sc, part_tc)
    return loss.reshape(())


# speedup vs baseline: 1.2919x; 1.0043x over previous
import functools

import jax
import jax.numpy as jnp
from jax import lax
from jax.experimental import pallas as pl
from jax.experimental.pallas import tpu as pltpu
from jax.experimental.pallas import tpu_sc as plsc

_E = 64
_Q = 1.2
_EPS = 1e-5
_ENT_TH = 2.5
_TOP_P = 0.75

_N = 65536
_NC, _NS, _L = 2, 16, 16
_NW = _NC * _NS
_W_SC = 8192
_TPW = _W_SC // _NW
_CH = 256
_NCH = _TPW // _CH
_NG = _CH // _L
_PR = 2 * _E + 4
_TB = 2048


def _sc_body(x_hbm, w_hbm, out_hbm, xb, wb, pbuf, acc):
    wid = lax.axis_index("s") * _NC + lax.axis_index("c")
    zeros = jnp.zeros((_L,), jnp.float32)

    def _zinit(j, carry):
        acc[j, :] = zeros
        return carry

    lax.fori_loop(0, _PR, _zinit, 0)

    def _chunk_body(std0):

        def _group(g, std):
            S, T, D = std
            sl = pl.ds(g * _L, _L)
            ninf = jnp.full((_L,), -1e30, jnp.float32)

            def _p1(blk, carry):
                m, m2x = carry
                for k in range(8):
                    x = xb[blk * 8 + k, sl]
                    m2x = jnp.maximum(m2x, jnp.minimum(m, x))
                    m = jnp.maximum(m, x)
                return m, m2x

            m, m2x = lax.fori_loop(0, 8, _p1, (ninf, ninf))

            def _p2(blk, z):
                for k in range(8):
                    j = blk * 8 + k
                    e = jnp.exp(xb[j, sl] - m)
                    pbuf[j, sl] = e
                    z = z + e
                return z

            z = lax.fori_loop(0, 8, _p2, zeros)
            rz = 1.0 / z

            y0 = jnp.where(z >= 20.0855, 3.5,
                           jnp.where(z >= 7.3891, 2.5,
                                     jnp.where(z >= 2.7183, 1.5, 0.5)))

            def _newton(_, y):
                return y - 1.0 + z * jnp.exp(-y)

            lnz = lax.fori_loop(0, 5, _newton, y0)
            k12 = _Q * (m + lnz)
            e2 = jnp.exp(m2x - m)

            def _p4(blk, carry):
                S, pqs = carry
                for k in range(8):
                    j = blk * 8 + k
                    ev = pbuf[j, sl]
                    S = S + jnp.maximum(ev * rz, _EPS)
                    pqs = pqs + jnp.maximum(jnp.exp(_Q * xb[j, sl] - k12),
                                            1e-6)
                return S, pqs

            S, pqs = lax.fori_loop(0, 8, _p4, (S, zeros))
            T = T + pqs
            high = ((1.0 - pqs) / (_Q - 1.0)) >= _ENT_TH

            thr = _TOP_P * z

            def _bis(_, lh):
                lo, hi = lh
                u = 0.5 * (lo + hi)

                def _gsum(blk, gs):
                    for k in range(8):
                        ev = pbuf[blk * 8 + k, sl]
                        gs = gs + jnp.where(ev >= u, ev, 0.0)
                    return gs

                gs = lax.fori_loop(0, 8, _gsum, zeros)
                ok = gs <= thr
                return jnp.where(ok, lo, u), jnp.where(ok, u, hi)

            _, hi = lax.fori_loop(
                0, 16, _bis, (zeros, jnp.full((_L,), 2.0, jnp.float32)))

            wv = wb[sl]
            rzw = rz * wv

            def _p6(blk, carry):
                for k in range(8):
                    j = blk * 8 + k
                    ev = pbuf[j, sl]
                    keep = high | (ev >= e2) | (ev >= hi)
                    rwv = jnp.where(keep, ev, 0.0)
                    plsc.addupdate(acc.at[j, :], rwv * rzw)
                    plsc.addupdate(acc.at[_E + j, :], ev * rzw)
                return carry

            lax.fori_loop(0, 8, _p6, 0)
            return (S, T, D + wv)

        return lax.fori_loop(0, _NG, _group, std0)

    std = (zeros, zeros, zeros)
    for c in range(_NCH):
        pltpu.sync_copy(x_hbm.at[wid, c], xb)
        pltpu.sync_copy(w_hbm.at[wid, c], wb)
        std = _chunk_body(std)
    S, T, D = std

    acc[2 * _E, :] = S
    acc[2 * _E + 1, :] = T
    acc[2 * _E + 2, :] = D
    acc[2 * _E + 3, :] = zeros
    pltpu.sync_copy(acc, out_hbm.at[wid])


def _tc_body(nb, x_ref, w_ref, out_ref, accA, accB, accSTD):
    b = pl.program_id(0)

    @pl.when(b == 0)
    def _init():
        accA[...] = jnp.zeros_like(accA)
        accB[...] = jnp.zeros_like(accB)
        accSTD[...] = jnp.zeros_like(accSTD)

    x = x_ref[...]
    w = w_ref[...]
    mx = jnp.max(x, axis=0, keepdims=True)
    e = jnp.exp(x - mx)
    z = jnp.sum(e, axis=0, keepdims=True)
    p = e / z

    pc = jnp.maximum(p, _EPS)
    pq = jnp.maximum(jnp.exp(_Q * (x - mx - jnp.log(z))), 1e-6)
    sum_pq_tok = jnp.sum(pq, axis=0, keepdims=True)
    ent = (1.0 - sum_pq_tok) / (_Q - 1.0)
    high = ent >= _ENT_TH

    m1 = jnp.max(p, axis=0, keepdims=True)
    m2 = jnp.max(jnp.where(p < m1, p, -1.0), axis=0, keepdims=True)

    lo = jnp.zeros_like(m1)
    hi = jnp.full_like(m1, 2.0)
    for _ in range(10):
        u = 0.5 * (lo + hi)
        gs = jnp.sum(jnp.where(p >= u, p, 0.0), axis=0, keepdims=True)
        ok = gs <= _TOP_P
        lo = jnp.where(ok, lo, u)
        hi = jnp.where(ok, u, hi)

    keep = high | (p >= m2) | (p >= hi)
    rw = jnp.where(keep, p, 0.0)

    accA[...] += jnp.sum(rw * w, axis=1, keepdims=True)
    accB[...] += jnp.sum(p * w, axis=1, keepdims=True)
    accSTD[0:1, :] += jnp.sum(pc)
    accSTD[1:2, :] += jnp.sum(pq)
    accSTD[2:3, :] += jnp.sum(w)

    @pl.when(b == nb - 1)
    def _fin():
        out_ref[0:_E, :] = accA[...]
        out_ref[_E:2 * _E, :] = accB[...]
        out_ref[2 * _E:2 * _E + 3, :] = accSTD[...]
        out_ref[2 * _E + 3:, :] = jnp.zeros_like(out_ref[2 * _E + 3:, :])


def _combine_body(p_sc_ref, p_tc_ref, out_ref):
    pm = jnp.sum(p_sc_ref[...], axis=0)
    q = p_tc_ref[...]
    a = jnp.sum(pm[0:_E, :], axis=1, keepdims=True) + q[0:_E, :]
    b = (jnp.sum(pm[_E:2 * _E, :], axis=1, keepdims=True)
         + q[_E:2 * _E, :])
    s = (jnp.sum(pm[2 * _E:2 * _E + 1, :], axis=1, keepdims=True)
         + q[2 * _E:2 * _E + 1, :])
    t = (jnp.sum(pm[2 * _E + 1:2 * _E + 2, :], axis=1, keepdims=True)
         + q[2 * _E + 1:2 * _E + 2, :])
    d = (jnp.sum(pm[2 * _E + 2:2 * _E + 3, :], axis=1, keepdims=True)
         + q[2 * _E + 2:2 * _E + 3, :])
    ent = (1.0 - t / (s ** _Q)) / (_Q - 1.0)
    lb = _E * jnp.sum(a * b, axis=0, keepdims=True) / (d * d)
    out_ref[...] = 0.001 * ent + 0.001 * lb


def kernel(gate_logits, attention_mask):
    n, e = gate_logits.shape
    bsz, seq = attention_mask.shape
    layers = n // (bsz * seq)

    wrow = jnp.broadcast_to(
        attention_mask.reshape(-1)[None, :], (layers, bsz * seq)
    ).reshape(1, n).astype(jnp.float32)

    x_r = gate_logits[:_W_SC].reshape(_NW, _NCH, _CH, e).transpose(0, 1, 3, 2)
    w_r = wrow[0, :_W_SC].reshape(_NW, _NCH, _CH)

    mesh = plsc.VectorSubcoreMesh(
        core_axis_name="c", subcore_axis_name="s",
        num_cores=_NC, num_subcores=_NS)
    partials_sc = pl.kernel(
        _sc_body,
        out_type=jax.ShapeDtypeStruct((_NW, _PR, _L), jnp.float32),
        mesh=mesh,
        scratch_types=[
            pltpu.VMEM((_E, _CH), jnp.float32),
            pltpu.VMEM((_CH,), jnp.float32),
            pltpu.VMEM((_E, _CH), jnp.float32),
            pltpu.VMEM((_PR, _L), jnp.float32),
        ],
    )(x_r, w_r)

    xt = gate_logits.T
    nb = (n - _W_SC) // _TB
    off = _W_SC // _TB
    part_tc = pl.pallas_call(
        functools.partial(_tc_body, nb),
        grid=(nb,),
        in_specs=[
            pl.BlockSpec((e, _TB), lambda i: (0, i + off)),
            pl.BlockSpec((1, _TB), lambda i: (0, i + off)),
        ],
        out_specs=pl.BlockSpec((_PR, 1), lambda i: (0, 0)),
        out_shape=jax.ShapeDtypeStruct((_PR, 1), jnp.float32),
        scratch_shapes=[
            pltpu.VMEM((_E, 1), jnp.float32),
            pltpu.VMEM((_E, 1), jnp.float32),
            pltpu.VMEM((3, 1), jnp.float32),
        ],
        compiler_params=pltpu.CompilerParams(
            dimension_semantics=("arbitrary",),
        ),
    )(xt, wrow)

    loss = pl.pallas_call(
        _combine_body,
        out_shape=jax.ShapeDtypeStruct((1, 1), jnp.float32),
    )(partials_sc, part_tc)
    return loss.reshape(())
